# baseline (speedup 1.0000x reference)
import functools

import jax
import jax.numpy as jnp
from jax import lax
from jax.experimental import pallas as pl
from jax.experimental.pallas import tpu as pltpu
from jax.experimental.pallas import tpu_sc as plsc

_D = 64
_B = 16384
_V = 1000000
_TPW = 244
_MAIN = _TPW * 32 * 128
_TAIL0 = _MAIN
_SLABC = 256
_NSLAB = _TPW * 128 // _SLABC
_TRASH = _B
_BP = _B + 16

_info = plsc.get_sparse_core_info()
_NC = _info.num_cores
_NS = _info.num_subcores
_NW = _NC * _NS

_mesh = plsc.VectorSubcoreMesh(core_axis_name="c", subcore_axis_name="s")


@functools.partial(
    pl.kernel,
    mesh=_mesh,
    out_type=jax.ShapeDtypeStruct((_BP, 128), jnp.float32),
    scratch_types=[
        pltpu.VMEM((_B + 16,), jnp.int32),
        pltpu.VMEM((_B + 16,), jnp.int32),
        pltpu.VMEM((_B + 16,), jnp.int32),
        pltpu.VMEM((_B + 16,), jnp.int32),
        pltpu.VMEM((_B + 16,), jnp.int32),
        pltpu.VMEM((2, _D, _SLABC), jnp.float32),
        pltpu.VMEM((2, 16, 128), jnp.float32),
        pltpu.SemaphoreType.DMA,
        pltpu.SemaphoreType.DMA,
    ],
    compiler_params=pltpu.CompilerParams(needs_layout_passes=False),
)
def _gather_kernel(table_hbm, tail_hbm, idx_hbm, out_hbm, idxb, mr, mb,
                   sr, sb, slab_v, stage_v, sem, osem):
    wid = lax.axis_index("s") * _NC + lax.axis_index("c")
    lo = wid * (_TPW * 128)
    is_last = wid == (_NW - 1)
    hi = jnp.where(is_last, _V, lo + _TPW * 128)

    pltpu.sync_copy(idx_hbm, idxb.at[pl.ds(0, _B)])

    lanes = lax.iota(jnp.int32, 16)
    zeros16 = jnp.full((16,), 0, jnp.int32)

    def scan_body(i, cnt):
        v = idxb[pl.ds(i * 16, 16)]
        m = jnp.logical_and(v >= lo, v < hi)
        plsc.store_compressed(mr.at[pl.ds(cnt, 16)], v, mask=m)
        plsc.store_compressed(mb.at[pl.ds(cnt, 16)], lanes + i * 16, mask=m)
        return cnt + jnp.max(plsc.all_reduce_population_count(m))

    cnt = lax.fori_loop(0, _B // 16, scan_body, jnp.int32(0))
    nwin = lax.shift_right_logical(cnt + 15, 4)

    def process_slab(slab_lo, slab_w, src_ref):
        def filt_body(i, c2):
            r = mr[pl.ds(i * 16, 16)]
            valid = (lanes + i * 16) < cnt
            m = jnp.logical_and(
                jnp.logical_and(r >= slab_lo, r < slab_lo + slab_w), valid
            )
            plsc.store_compressed(sr.at[pl.ds(c2, 16)], r, mask=m)
            plsc.store_compressed(
                sb.at[pl.ds(c2, 16)], mb[pl.ds(i * 16, 16)], mask=m
            )
            return c2 + jnp.max(plsc.all_reduce_population_count(m))

        c2 = lax.fori_loop(0, nwin, filt_body, jnp.int32(0))
        nwin2 = lax.shift_right_logical(c2 + 15, 4)

        def win_body(w, carry):
            r = sr[pl.ds(w * 16, 16)]
            valid = (lanes + w * 16) < c2
            bsafe = jnp.where(valid, sb[pl.ds(w * 16, 16)], _TRASH)
            m = jnp.minimum(
                jnp.maximum(r - slab_lo, zeros16), slab_w - 1
            )
            buf = lax.rem(w, 2)

            @pl.when(w >= 2)
            def _():
                pltpu.make_async_copy(
                    out_hbm.at[pl.ds(0, 16)], stage_v.at[buf], osem
                ).wait()

            for c in range(_D):
                cvec = zeros16 + c
                vals = plsc.load_gather(src_ref, [cvec, m])
                plsc.store_scatter(stage_v.at[buf], [lanes, cvec], vals)
            pltpu.async_copy(stage_v.at[buf], out_hbm.at[bsafe], osem)
            return carry

        lax.fori_loop(0, nwin2, win_body, jnp.int32(0))

        def drain_body(i, carry):
            pltpu.make_async_copy(
                out_hbm.at[pl.ds(0, 16)], stage_v.at[0], osem
            ).wait()
            return carry

        lax.fori_loop(0, jnp.minimum(nwin2, 2), drain_body, jnp.int32(0))

    def slab_src(s):
        return table_hbm.at[:, pl.ds(lo + s * _SLABC, _SLABC)]

    pltpu.async_copy(slab_src(0), slab_v.at[0], sem)

    def stream_body(s, carry):
        buf = lax.rem(s, 2)

        @pl.when(s + 1 < _NSLAB)
        def _():
            pltpu.async_copy(slab_src(s + 1), slab_v.at[1 - buf], sem)

        pltpu.make_async_copy(slab_src(s), slab_v.at[buf], sem).wait()
        process_slab(lo + s * _SLABC, _SLABC, slab_v.at[buf])
        return carry

    lax.fori_loop(0, _NSLAB, stream_body, jnp.int32(0))

    @pl.when(is_last)
    def _():
        for t in range(2):
            pltpu.sync_copy(
                table_hbm.at[:, pl.ds(_TAIL0 + t * _SLABC, _SLABC)],
                slab_v.at[0],
            )
            process_slab(_TAIL0 + t * _SLABC, _SLABC, slab_v.at[0])
        pltpu.sync_copy(tail_hbm, slab_v.at[0, :, pl.ds(0, 128)])
        process_slab(_TAIL0 + 2 * _SLABC, _D, slab_v.at[0])


def kernel(inputs, in_embed_weight):
    idx = inputs.astype(jnp.int32)
    tail = jnp.pad(
        in_embed_weight[_TAIL0 + 2 * _SLABC:].T, ((0, 0), (0, 128 - _D))
    )
    out_p = _gather_kernel(in_embed_weight.T, tail, idx)
    return out_p[:_B, :_D]


# speedup vs baseline: 2.2989x; 2.2989x over previous
import functools

import jax
import jax.numpy as jnp
from jax import lax
from jax.experimental import pallas as pl
from jax.experimental.pallas import tpu as pltpu
from jax.experimental.pallas import tpu_sc as plsc

_D = 64
_B = 16384
_V = 1000000
_TPW = 244
_MAIN = _TPW * 32 * 128
_TAIL0 = _MAIN
_SLABC = 512
_NSLAB = _TPW * 128 // _SLABC
_TRASH = _B
_BP = _B + 16

_info = plsc.get_sparse_core_info()
_NC = _info.num_cores
_NS = _info.num_subcores
_NW = _NC * _NS

_mesh = plsc.VectorSubcoreMesh(core_axis_name="c", subcore_axis_name="s")


@functools.partial(
    pl.kernel,
    mesh=_mesh,
    out_type=jax.ShapeDtypeStruct((_BP, 128), jnp.float32),
    scratch_types=[
        pltpu.VMEM((_B + 16,), jnp.int32),
        pltpu.VMEM((_B + 16,), jnp.int32),
        pltpu.VMEM((2, _D, _SLABC), jnp.float32),
        pltpu.VMEM((2, 16, 128), jnp.float32),
        pltpu.SemaphoreType.DMA,
        pltpu.SemaphoreType.DMA,
    ],
    compiler_params=pltpu.CompilerParams(needs_layout_passes=False),
)
def _gather_kernel(table_hbm, tail_hbm, idx_hbm, out_hbm, idxb, mp,
                   slab_v, stage_v, sem, osem):
    wid = lax.axis_index("s") * _NC + lax.axis_index("c")
    lo = wid * (_TPW * 128)
    is_last = wid == (_NW - 1)
    hi = jnp.where(is_last, _V, lo + _TPW * 128)
    sp = idxb

    pltpu.sync_copy(idx_hbm, idxb.at[pl.ds(0, _B)])

    lanes = lax.iota(jnp.int32, 16)
    zeros16 = jnp.full((16,), 0, jnp.int32)

    def scan_body(i, cnt):
        v = idxb[pl.ds(i * 16, 16)]
        m = jnp.logical_and(v >= lo, v < hi)
        packed = lax.shift_left(v - lo, 14) + (lanes + i * 16)
        plsc.store_compressed(mp.at[pl.ds(cnt, 16)], packed, mask=m)
        return cnt + jnp.max(plsc.all_reduce_population_count(m))

    cnt = lax.fori_loop(0, _B // 16, scan_body, jnp.int32(0))
    nwin = lax.shift_right_logical(cnt + 15, 4)

    def process_slab(rel_lo, slab_w, src_ref):
        def filt_body(i, c2):
            p = mp[pl.ds(i * 16, 16)]
            r = lax.shift_right_logical(p, 14)
            valid = (lanes + i * 16) < cnt
            m = jnp.logical_and(
                jnp.logical_and(r >= rel_lo, r < rel_lo + slab_w), valid
            )
            plsc.store_compressed(sp.at[pl.ds(c2, 16)], p, mask=m)
            return c2 + jnp.max(plsc.all_reduce_population_count(m))

        c2 = lax.fori_loop(0, nwin, filt_body, jnp.int32(0))
        nwin2 = lax.shift_right_logical(c2 + 15, 4)

        def win_body(w, carry):
            p = sp[pl.ds(w * 16, 16)]
            r = lax.shift_right_logical(p, 14)
            valid = (lanes + w * 16) < c2
            bsafe = jnp.where(
                valid, lax.bitwise_and(p, (1 << 14) - 1), _TRASH
            )
            m = jnp.minimum(
                jnp.maximum(r - rel_lo, zeros16), slab_w - 1
            )
            buf = lax.rem(w, 2)

            @pl.when(w >= 2)
            def _():
                pltpu.make_async_copy(
                    out_hbm.at[pl.ds(0, 16)], stage_v.at[buf], osem
                ).wait()

            for c in range(_D):
                cvec = zeros16 + c
                vals = plsc.load_gather(src_ref, [cvec, m])
                plsc.store_scatter(stage_v.at[buf], [lanes, cvec], vals)
            pltpu.async_copy(stage_v.at[buf], out_hbm.at[bsafe], osem)
            return carry

        lax.fori_loop(0, nwin2, win_body, jnp.int32(0))

        def drain_body(i, carry):
            pltpu.make_async_copy(
                out_hbm.at[pl.ds(0, 16)], stage_v.at[0], osem
            ).wait()
            return carry

        lax.fori_loop(0, jnp.minimum(nwin2, 2), drain_body, jnp.int32(0))

    def slab_start(s, buf):
        col = lo + s * _SLABC
        for g in range(_D // 8):
            pltpu.async_copy(
                table_hbm.at[pl.ds(g * 8, 8), pl.ds(col, _SLABC)],
                slab_v.at[buf, pl.ds(g * 8, 8), :],
                sem,
            )

    def slab_wait(s, buf):
        col = lo + s * _SLABC
        for g in range(_D // 8):
            pltpu.make_async_copy(
                table_hbm.at[pl.ds(g * 8, 8), pl.ds(col, _SLABC)],
                slab_v.at[buf, pl.ds(g * 8, 8), :],
                sem,
            ).wait()

    slab_start(0, 0)

    def stream_body(s, carry):
        buf = lax.rem(s, 2)

        @pl.when(s + 1 < _NSLAB)
        def _():
            slab_start(s + 1, 1 - buf)

        slab_wait(s, buf)
        process_slab(s * _SLABC, _SLABC, slab_v.at[buf])
        return carry

    lax.fori_loop(0, _NSLAB, stream_body, jnp.int32(0))

    @pl.when(is_last)
    def _():
        for g in range(_D // 8):
            pltpu.sync_copy(
                table_hbm.at[pl.ds(g * 8, 8), pl.ds(_TAIL0, 512)],
                slab_v.at[0, pl.ds(g * 8, 8), :],
            )
        process_slab(_TAIL0 - lo, 512, slab_v.at[0])
        pltpu.sync_copy(tail_hbm, slab_v.at[0, :, pl.ds(0, 128)])
        process_slab(_TAIL0 + 512 - lo, _D, slab_v.at[0])


def kernel(inputs, in_embed_weight):
    idx = inputs.astype(jnp.int32)
    tail = jnp.pad(
        in_embed_weight[_TAIL0 + 512:].T, ((0, 0), (0, 128 - _D))
    )
    out_p = _gather_kernel(in_embed_weight.T, tail, idx)
    return out_p[:_B, :_D]
